# baseline (speedup 1.0000x reference)
import functools
import jax
import jax.numpy as jnp
from jax import lax
from jax.experimental import pallas as pl
from jax.experimental.pallas import tpu as pltpu
from jax.experimental.pallas import tpu_sc as plsc

N, D, E, H = 2048, 1024, 8, 4096
BLK = 256
HB = 1024
NB = N // BLK
NH = H // HB

NC, NS, L = 2, 16, 16
CH = 64
DH = D // NC
ACC_ROWS = N + 128

_mesh = plsc.VectorSubcoreMesh(core_axis_name="c", subcore_axis_name="s")


@functools.partial(
    pl.kernel, mesh=_mesh,
    compiler_params=pltpu.CompilerParams(needs_layout_passes=False),
    out_type=[
        jax.ShapeDtypeStruct((E * N,), jnp.int32),
        jax.ShapeDtypeStruct((E * N,), jnp.float32),
        jax.ShapeDtypeStruct((E * L,), jnp.int32),
        jax.ShapeDtypeStruct((E * N,), jnp.int32),
    ],
    scratch_types=[
        pltpu.VMEM((N,), jnp.float32),
        pltpu.VMEM((N,), jnp.float32),
        pltpu.VMEM((N,), jnp.int32),
        pltpu.VMEM((N,), jnp.float32),
        pltpu.VMEM((L,), jnp.int32),
        pltpu.VMEM((N,), jnp.int32),
    ],
)
def _sc_compact(dispT, combT, idx_out, coef_out, cnt_out, pos_out,
                disp_v, comb_v, idx_v, coef_v, cnt_v, pos_v):
    c = lax.axis_index("c")
    s = lax.axis_index("s")
    wid = s * NC + c

    @pl.when(wid < E)
    def _():
        e0 = pl.multiple_of(wid * N, N)
        pltpu.sync_copy(dispT.at[pl.ds(e0, N)], disp_v)
        pltpu.sync_copy(combT.at[pl.ds(e0, N)], comb_v)

        def zbody(i, _):
            idx_v[pl.ds(i * L, L)] = jnp.zeros((L,), jnp.int32)
            coef_v[pl.ds(i * L, L)] = jnp.zeros((L,), jnp.float32)
            return 0
        lax.fori_loop(0, N // L, zbody, 0)

        def pcount(i, acc_vec):
            dv = disp_v[pl.ds(i * L, L)]
            return acc_vec + plsc.all_reduce_population_count(dv > 0.0)
        cnt_vec = lax.fori_loop(0, N // L, pcount, jnp.zeros((L,), jnp.int32))

        def pinit(i, _):
            pos_v[pl.ds(i * L, L)] = cnt_vec + e0
            return 0
        lax.fori_loop(0, N // L, pinit, 0)

        def body(i, off_vec):
            dv = disp_v[pl.ds(i * L, L)]
            m = dv > 0.0
            cm = jnp.where(m, jnp.full((L,), 1, jnp.int32), jnp.zeros((L,), jnp.int32))
            pos = off_vec + plsc.cumsum(cm) - 1
            tok = lax.iota(jnp.int32, L) + i * L
            plsc.store_scatter(idx_v, [pos], tok, mask=m)
            cb = comb_v[pl.ds(i * L, L)]
            plsc.store_scatter(coef_v, [pos], cb, mask=m)
            plsc.store_scatter(pos_v, [tok], pos + e0, mask=m)
            return off_vec + plsc.all_reduce_population_count(m)
        lax.fori_loop(0, N // L, body, jnp.zeros((L,), jnp.int32))

        pltpu.sync_copy(idx_v, idx_out.at[pl.ds(e0, N)])
        pltpu.sync_copy(coef_v, coef_out.at[pl.ds(e0, N)])
        cnt_v[...] = cnt_vec
        pltpu.sync_copy(cnt_v, cnt_out.at[pl.ds(pl.multiple_of(wid * L, L), L)])
        pltpu.sync_copy(pos_v, pos_out.at[pl.ds(e0, N)])


_QB = N // 4

@functools.partial(
    pl.kernel, mesh=_mesh,
    compiler_params=pltpu.CompilerParams(needs_layout_passes=False),
    out_type=jax.ShapeDtypeStruct((E * N, D), jnp.float32),
    scratch_types=[
        pltpu.VMEM((N,), jnp.float32),
        pltpu.VMEM((CH,), jnp.int32),
        pltpu.VMEM((CH, D), jnp.float32),
        pltpu.SemaphoreType.DMA,
    ],
)
def _sc_gather(flat, dispT, idx, xg_out, disp_v, idxc, rows, sem):
    c = lax.axis_index("c")
    s = lax.axis_index("s")
    wid = s * NC + c
    e = wid // 4
    q = wid % 4

    e0 = pl.multiple_of(e * N, N)
    pltpu.sync_copy(dispT.at[pl.ds(e0, N)], disp_v)

    def cbody(i, acc_vec):
        dv = disp_v[pl.ds(i * L, L)]
        return acc_vec + plsc.all_reduce_population_count(dv > 0.0)
    cnt_vec = lax.fori_loop(0, N // L, cbody, jnp.zeros((L,), jnp.int32))
    cnt = cnt_vec[0]

    lo = q * _QB
    span = jnp.maximum(jnp.minimum(cnt, lo + _QB) - lo, 0)
    nch = (span + CH - 1) // CH

    def gbody(jc, _):
        base = pl.multiple_of(e0 + lo + jc * CH, CH)
        pltpu.sync_copy(idx.at[pl.ds(base, CH)], idxc)
        pltpu.async_copy(flat.at[idxc], rows, sem).wait()
        pltpu.sync_copy(rows, xg_out.at[pl.ds(base, CH), :])
        return 0
    lax.fori_loop(0, nch, gbody, 0)


def _ffn_kernel(c_ref, x_ref, wg_ref, wv_ref, wo_ref, coef_ref, scale_ref, y_ref):
    e = pl.program_id(0)
    h = pl.program_id(1)
    j = pl.program_id(2)
    cnt = c_ref[e]

    @pl.when(j * BLK < cnt)
    def _active():
        x = x_ref[...]
        wg = wg_ref[0]
        wv = wv_ref[0]
        wo = wo_ref[0]

        gate = jax.lax.dot_general(x, wg, (((1,), (1,)), ((), ())),
                                   preferred_element_type=jnp.float32)
        gate = gate * 0.5 * (1.0 + jax.lax.erf(gate * 0.7071067811865476))
        value = jax.lax.dot_general(x, wv, (((1,), (1,)), ((), ())),
                                    preferred_element_type=jnp.float32)
        hidden = gate * value
        part = jax.lax.dot_general(hidden, wo, (((1,), (1,)), ((), ())),
                                   preferred_element_type=jnp.float32)

        srow = jax.lax.broadcasted_iota(jnp.int32, (1, E), 1) == e
        scale_e = jnp.sum(jnp.where(srow, scale_ref[...], 0.0))
        contrib = part * (coef_ref[...] * scale_e)

        rows = pl.ds(j * BLK, BLK)

        @pl.when(h == 0)
        def _init():
            y_ref[rows, :] = contrib

        @pl.when(h != 0)
        def _acc():
            y_ref[rows, :] += contrib

    @pl.when(jnp.logical_and(jnp.logical_and(j * BLK >= cnt, h == 0),
                             j == (cnt + BLK - 1) // BLK))
    def _zero_first_inactive():
        y_ref[pl.ds(j * BLK, BLK), :] = jnp.zeros((BLK, D), jnp.float32)


def _jmax(c):
    return jnp.maximum((c + BLK - 1) // BLK - 1, 0)


def _ragged_ffn(counts, Xg, coefc, Wg, Wv, Wo, scale):
    return pl.pallas_call(
        _ffn_kernel,
        grid_spec=pltpu.PrefetchScalarGridSpec(
            num_scalar_prefetch=1,
            grid=(E, NH, NB),
            in_specs=[
                pl.BlockSpec((BLK, D),
                             lambda e, h, j, c: (e * NB + jnp.minimum(j, _jmax(c[e])), 0)),
                pl.BlockSpec((1, HB, D), lambda e, h, j, c: (e, h, 0)),
                pl.BlockSpec((1, HB, D), lambda e, h, j, c: (e, h, 0)),
                pl.BlockSpec((1, D, HB), lambda e, h, j, c: (e, 0, h)),
                pl.BlockSpec((BLK, 1),
                             lambda e, h, j, c: (e * NB + jnp.minimum(j, _jmax(c[e])), 0)),
                pl.BlockSpec((1, E), lambda e, h, j, c: (0, 0)),
            ],
            out_specs=pl.BlockSpec((N, D), lambda e, h, j, c: (e, 0)),
        ),
        out_shape=jax.ShapeDtypeStruct((E * N, D), jnp.float32),
    )(counts, Xg, Wg, Wv, Wo, coefc, scale.reshape(1, E))


_TT = N // (NC * NS)

@functools.partial(
    pl.kernel, mesh=_mesh,
    compiler_params=pltpu.CompilerParams(needs_layout_passes=False),
    out_type=jax.ShapeDtypeStruct((E * N, D), jnp.float32),
    scratch_types=[
        pltpu.VMEM((E * _TT,), jnp.int32),
        pltpu.VMEM((_TT, D), jnp.float32),
        pltpu.SemaphoreType.DMA,
        pltpu.SemaphoreType.DMA,
    ],
)
def _sc_deinterleave(pos, y, z_out, pbuf, zbuf, sem, wsem):
    c = lax.axis_index("c")
    s = lax.axis_index("s")
    wid = s * NC + c
    t0 = pl.multiple_of(wid * _TT, _TT)

    for e_ in range(E):
        pltpu.sync_copy(pos.at[pl.ds(e_ * N + t0, _TT)],
                        pbuf.at[pl.ds(e_ * _TT, _TT)])
    hw = _TT // 2
    prev_w = [None, None]
    for step in range(E * 2):
        e_, h = step // 2, step % 2
        half = step % 2
        if prev_w[half] is not None:
            prev_w[half].wait()
        gcp = pltpu.async_copy(
            y.at[pbuf.at[pl.ds(e_ * _TT + h * hw, hw)]],
            zbuf.at[pl.ds(half * hw, hw), :], sem)
        gcp.wait()
        prev_w[half] = pltpu.async_copy(
            zbuf.at[pl.ds(half * hw, hw), :],
            z_out.at[pl.ds(e_ * N + t0 + h * hw, hw), :], wsem)
    prev_w[0].wait()
    prev_w[1].wait()


def _sum_kernel(z_ref, o_ref):
    acc = z_ref[0]
    for k in range(1, E):
        acc = acc + z_ref[k]
    o_ref[...] = acc


def _tc_sum(Z):
    return pl.pallas_call(
        _sum_kernel,
        grid=(N // BLK,),
        in_specs=[pl.BlockSpec((E, BLK, D), lambda i: (0, i, 0))],
        out_specs=pl.BlockSpec((BLK, D), lambda i: (i, 0)),
        out_shape=jax.ShapeDtypeStruct((N, D), jnp.float32),
    )(Z)


def kernel(tokens, dispatch_weights, combine_weights, Wg, Wv, Wo, scale):
    b, n, d = tokens.shape
    flat = tokens.reshape(n, d)
    dispT = dispatch_weights.reshape(n, E).T.reshape(E * N)
    combT = combine_weights.reshape(n, E).T.reshape(E * N)

    idx, coefc, cnts, pos = _sc_compact(dispT, combT)
    counts = cnts.reshape(E, L)[:, 0]
    Xg = _sc_gather(flat, dispT, idx)
    Y = _ragged_ffn(counts, Xg, coefc.reshape(E * N, 1), Wg, Wv, Wo, scale)
    Z = _sc_deinterleave(pos, Y)
    out = _tc_sum(Z.reshape(E, N, D))
    return out.reshape(b, n, d)


# speedup vs baseline: 1.0473x; 1.0473x over previous
import functools
import jax
import jax.numpy as jnp
from jax import lax
from jax.experimental import pallas as pl
from jax.experimental.pallas import tpu as pltpu
from jax.experimental.pallas import tpu_sc as plsc

N, D, E, H = 2048, 1024, 8, 4096
BLK = 256
HB = 1024
NB = N // BLK
NH = H // HB

NC, NS, L = 2, 16, 16
CH = 64
DH = D // NC
ACC_ROWS = N + 128

_mesh = plsc.VectorSubcoreMesh(core_axis_name="c", subcore_axis_name="s")


@functools.partial(
    pl.kernel, mesh=_mesh,
    compiler_params=pltpu.CompilerParams(needs_layout_passes=False),
    out_type=[
        jax.ShapeDtypeStruct((E * N,), jnp.int32),
        jax.ShapeDtypeStruct((E * N,), jnp.float32),
        jax.ShapeDtypeStruct((E * L,), jnp.int32),
        jax.ShapeDtypeStruct((E * N,), jnp.int32),
    ],
    scratch_types=[
        pltpu.VMEM((N,), jnp.float32),
        pltpu.VMEM((N,), jnp.float32),
        pltpu.VMEM((N,), jnp.int32),
        pltpu.VMEM((N,), jnp.float32),
        pltpu.VMEM((L,), jnp.int32),
        pltpu.VMEM((N,), jnp.int32),
    ],
)
def _sc_compact(dispT, combT, idx_out, coef_out, cnt_out, pos_out,
                disp_v, comb_v, idx_v, coef_v, cnt_v, pos_v):
    c = lax.axis_index("c")
    s = lax.axis_index("s")
    wid = s * NC + c

    @pl.when(wid < E)
    def _():
        e0 = pl.multiple_of(wid * N, N)
        pltpu.sync_copy(dispT.at[pl.ds(e0, N)], disp_v)
        pltpu.sync_copy(combT.at[pl.ds(e0, N)], comb_v)

        def zbody(i, _):
            idx_v[pl.ds(i * L, L)] = jnp.zeros((L,), jnp.int32)
            coef_v[pl.ds(i * L, L)] = jnp.zeros((L,), jnp.float32)
            return 0
        lax.fori_loop(0, N // L, zbody, 0)

        def pcount(i, acc_vec):
            dv = disp_v[pl.ds(i * L, L)]
            return acc_vec + plsc.all_reduce_population_count(dv > 0.0)
        cnt_vec = lax.fori_loop(0, N // L, pcount, jnp.zeros((L,), jnp.int32))

        def pinit(i, _):
            pos_v[pl.ds(i * L, L)] = cnt_vec + e0
            return 0
        lax.fori_loop(0, N // L, pinit, 0)

        def body(i, off_vec):
            dv = disp_v[pl.ds(i * L, L)]
            m = dv > 0.0
            cm = jnp.where(m, jnp.full((L,), 1, jnp.int32), jnp.zeros((L,), jnp.int32))
            pos = off_vec + plsc.cumsum(cm) - 1
            tok = lax.iota(jnp.int32, L) + i * L
            plsc.store_scatter(idx_v, [pos], tok, mask=m)
            cb = comb_v[pl.ds(i * L, L)]
            plsc.store_scatter(coef_v, [pos], cb, mask=m)
            plsc.store_scatter(pos_v, [tok], pos + e0, mask=m)
            return off_vec + plsc.all_reduce_population_count(m)
        lax.fori_loop(0, N // L, body, jnp.zeros((L,), jnp.int32))

        pltpu.sync_copy(idx_v, idx_out.at[pl.ds(e0, N)])
        pltpu.sync_copy(coef_v, coef_out.at[pl.ds(e0, N)])
        cnt_v[...] = cnt_vec
        pltpu.sync_copy(cnt_v, cnt_out.at[pl.ds(pl.multiple_of(wid * L, L), L)])
        pltpu.sync_copy(pos_v, pos_out.at[pl.ds(e0, N)])


_QB = N // 4

@functools.partial(
    pl.kernel, mesh=_mesh,
    compiler_params=pltpu.CompilerParams(needs_layout_passes=False),
    out_type=jax.ShapeDtypeStruct((E * N, D), jnp.float32),
    scratch_types=[
        pltpu.VMEM((N,), jnp.float32),
        pltpu.VMEM((CH,), jnp.int32),
        pltpu.VMEM((CH, D), jnp.float32),
        pltpu.SemaphoreType.DMA,
    ],
)
def _sc_gather(flat, dispT, idx, xg_out, disp_v, idxc, rows, sem):
    c = lax.axis_index("c")
    s = lax.axis_index("s")
    wid = s * NC + c
    e = wid // 4
    q = wid % 4

    e0 = pl.multiple_of(e * N, N)
    pltpu.sync_copy(dispT.at[pl.ds(e0, N)], disp_v)

    def cbody(i, acc_vec):
        dv = disp_v[pl.ds(i * L, L)]
        return acc_vec + plsc.all_reduce_population_count(dv > 0.0)
    cnt_vec = lax.fori_loop(0, N // L, cbody, jnp.zeros((L,), jnp.int32))
    cnt = cnt_vec[0]

    lo = q * _QB
    span = jnp.maximum(jnp.minimum(cnt, lo + _QB) - lo, 0)
    nch = (span + CH - 1) // CH

    def gbody(jc, _):
        base = pl.multiple_of(e0 + lo + jc * CH, CH)
        pltpu.sync_copy(idx.at[pl.ds(base, CH)], idxc)
        pltpu.async_copy(flat.at[idxc], rows, sem).wait()
        pltpu.sync_copy(rows, xg_out.at[pl.ds(base, CH), :])
        return 0
    lax.fori_loop(0, nch, gbody, 0)


def _ffn_kernel(c_ref, x_ref, wg_ref, wv_ref, wo_ref, coef_ref, scale_ref, y_ref):
    e = pl.program_id(0)
    h = pl.program_id(1)
    j = pl.program_id(2)
    cnt = c_ref[e]

    @pl.when(j * BLK < cnt)
    def _active():
        x = x_ref[...]
        wg = wg_ref[0]
        wv = wv_ref[0]
        wo = wo_ref[0]

        gate = jax.lax.dot_general(x, wg, (((1,), (1,)), ((), ())),
                                   preferred_element_type=jnp.float32)
        gate = gate * 0.5 * (1.0 + jax.lax.erf(gate * 0.7071067811865476))
        value = jax.lax.dot_general(x, wv, (((1,), (1,)), ((), ())),
                                    preferred_element_type=jnp.float32)
        hidden = gate * value
        part = jax.lax.dot_general(hidden, wo, (((1,), (1,)), ((), ())),
                                   preferred_element_type=jnp.float32)

        srow = jax.lax.broadcasted_iota(jnp.int32, (1, E), 1) == e
        scale_e = jnp.sum(jnp.where(srow, scale_ref[...], 0.0))
        contrib = part * (coef_ref[...] * scale_e)

        rows = pl.ds(j * BLK, BLK)

        @pl.when(h == 0)
        def _init():
            y_ref[rows, :] = contrib

        @pl.when(h != 0)
        def _acc():
            y_ref[rows, :] += contrib

    @pl.when(jnp.logical_and(jnp.logical_and(j * BLK >= cnt, h == 0),
                             j == (cnt + BLK - 1) // BLK))
    def _zero_first_inactive():
        y_ref[pl.ds(j * BLK, BLK), :] = jnp.zeros((BLK, D), jnp.float32)


def _jmax(c):
    return jnp.maximum((c + BLK - 1) // BLK - 1, 0)


def _ragged_ffn(counts, Xg, coefc, Wg, Wv, Wo, scale):
    return pl.pallas_call(
        _ffn_kernel,
        grid_spec=pltpu.PrefetchScalarGridSpec(
            num_scalar_prefetch=1,
            grid=(E, NH, NB),
            in_specs=[
                pl.BlockSpec((BLK, D),
                             lambda e, h, j, c: (e * NB + jnp.minimum(j, _jmax(c[e])), 0)),
                pl.BlockSpec((1, HB, D), lambda e, h, j, c: (e, h, 0)),
                pl.BlockSpec((1, HB, D), lambda e, h, j, c: (e, h, 0)),
                pl.BlockSpec((1, D, HB), lambda e, h, j, c: (e, 0, h)),
                pl.BlockSpec((BLK, 1),
                             lambda e, h, j, c: (e * NB + jnp.minimum(j, _jmax(c[e])), 0)),
                pl.BlockSpec((1, E), lambda e, h, j, c: (0, 0)),
            ],
            out_specs=pl.BlockSpec((N, D), lambda e, h, j, c: (e, 0)),
        ),
        out_shape=jax.ShapeDtypeStruct((E * N, D), jnp.float32),
    )(counts, Xg, Wg, Wv, Wo, coefc, scale.reshape(1, E))


_TT = N // (NC * NS)

@functools.partial(
    pl.kernel, mesh=_mesh,
    compiler_params=pltpu.CompilerParams(needs_layout_passes=False),
    out_type=jax.ShapeDtypeStruct((E * N, D), jnp.float32),
    scratch_types=[
        pltpu.VMEM((E * _TT,), jnp.int32),
        pltpu.VMEM((_TT, D), jnp.float32),
        pltpu.SemaphoreType.DMA,
        pltpu.SemaphoreType.DMA,
    ],
)
def _sc_deinterleave(pos, y, z_out, pbuf, zbuf, sem, wsem):
    c = lax.axis_index("c")
    s = lax.axis_index("s")
    wid = s * NC + c
    t0 = pl.multiple_of(wid * _TT, _TT)

    for e_ in range(E):
        pltpu.sync_copy(pos.at[pl.ds(e_ * N + t0, _TT)],
                        pbuf.at[pl.ds(e_ * _TT, _TT)])
    qr = L
    gw = [None] * 4
    for e_ in range(E):
        gcs = []
        for sl in range(4):
            if gw[sl] is not None:
                gw[sl].wait()
            gcs.append(pltpu.async_copy(
                y.at[pbuf.at[pl.ds(e_ * _TT + sl * qr, qr)]],
                zbuf.at[pl.ds(sl * qr, qr), :], sem))
        for sl in range(4):
            gcs[sl].wait()
            gw[sl] = pltpu.async_copy(
                zbuf.at[pl.ds(sl * qr, qr), :],
                z_out.at[pl.ds(e_ * N + t0 + sl * qr, qr), :], wsem)
    for sl in range(4):
        gw[sl].wait()


def _sum_kernel(z_ref, o_ref):
    acc = z_ref[0]
    for k in range(1, E):
        acc = acc + z_ref[k]
    o_ref[...] = acc


def _tc_sum(Z):
    return pl.pallas_call(
        _sum_kernel,
        grid=(N // BLK,),
        in_specs=[pl.BlockSpec((E, BLK, D), lambda i: (0, i, 0))],
        out_specs=pl.BlockSpec((BLK, D), lambda i: (i, 0)),
        out_shape=jax.ShapeDtypeStruct((N, D), jnp.float32),
    )(Z)


def kernel(tokens, dispatch_weights, combine_weights, Wg, Wv, Wo, scale):
    b, n, d = tokens.shape
    flat = tokens.reshape(n, d)
    dispT = dispatch_weights.reshape(n, E).T.reshape(E * N)
    combT = combine_weights.reshape(n, E).T.reshape(E * N)

    idx, coefc, cnts, pos = _sc_compact(dispT, combT)
    counts = cnts.reshape(E, L)[:, 0]
    Xg = _sc_gather(flat, dispT, idx)
    Y = _ragged_ffn(counts, Xg, coefc.reshape(E * N, 1), Wg, Wv, Wo, scale)
    Z = _sc_deinterleave(pos, Y)
    out = _tc_sum(Z.reshape(E, N, D))
    return out.reshape(b, n, d)
